# baseline (speedup 1.0000x reference)
import functools

import jax
import jax.numpy as jnp
from jax.experimental import pallas as pl
from jax.experimental.pallas import tpu as pltpu


def _chunk_compute(xx, w1b, b1, w2b, b2, ones):
    h = jax.lax.dot_general(
        xx.astype(jnp.bfloat16), w1b, (((1,), (1,)), ((), ())),
        preferred_element_type=jnp.float32,
    )
    h = (h + b1).astype(jnp.bfloat16)
    h = jnp.where(h > 0, h, jnp.exp(h) - 1.0)
    h = jax.lax.dot_general(
        h, w2b, (((1,), (1,)), ((), ())),
        preferred_element_type=jnp.float32,
    ) + b2
    h = jnp.where(h > 0, h, jnp.exp(h) - 1.0)
    m = jnp.max(h, axis=1, keepdims=True)
    s = h - m
    e = jnp.exp(s).astype(jnp.bfloat16)
    sums = jax.lax.dot_general(
        e, ones, (((1,), (0,)), ((), ())),
        preferred_element_type=jnp.float32,
    )
    lse = jnp.log(sums[:, :1])
    return s - lse


def _mlp_kernel(
    x_h, w1_h, b1_h, w2_h, b2_h, o_h,
    xbuf, obuf, w1_v, b1_v, w2_v, b2_v, sx, so, sw,
    *, chunks,
):
    wc = [
        pltpu.make_async_copy(w1_h, w1_v, sw.at[0]),
        pltpu.make_async_copy(b1_h, b1_v, sw.at[1]),
        pltpu.make_async_copy(w2_h, w2_v, sw.at[2]),
        pltpu.make_async_copy(b2_h, b2_v, sw.at[3]),
    ]
    for c in wc:
        c.start()
    xc = [
        pltpu.make_async_copy(
            x_h.at[pl.ds(off, sz), :], xbuf.at[i, pl.ds(0, sz)], sx.at[i]
        )
        for i, (off, sz) in enumerate(chunks)
    ]
    for c in xc:
        c.start()
    for c in wc:
        c.wait()
    w1b = w1_v[...].astype(jnp.bfloat16)
    w2b = w2_v[...].astype(jnp.bfloat16)
    b1 = b1_v[...]
    b2 = b2_v[...]
    ones = jnp.ones((64, 128), dtype=jnp.bfloat16)
    for i, ((off, sz), c) in enumerate(zip(chunks, xc)):
        c.wait()
        r = _chunk_compute(xbuf[i, :sz], w1b, b1, w2b, b2, ones)
        obuf[:, pl.ds(off, sz)] = r.T
    oc = pltpu.make_async_copy(obuf, o_h, so.at[0])
    oc.start()
    oc.wait()


def kernel(x, edge_index, W1, b1, W2, b2):
    N, D = x.shape
    H = W1.shape[0]
    C = W2.shape[0]
    CH = 1024
    chunks = []
    off = 0
    while off < N:
        sz = min(CH, N - off)
        chunks.append((off, sz))
        off += sz
    nc = len(chunks)
    hbm = pl.BlockSpec(memory_space=pltpu.MemorySpace.HBM)
    out_t = pl.pallas_call(
        functools.partial(_mlp_kernel, chunks=tuple(chunks)),
        in_specs=[hbm] * 5,
        out_specs=hbm,
        out_shape=jax.ShapeDtypeStruct((C, N), jnp.float32),
        scratch_shapes=[
            pltpu.VMEM((nc, CH, D), jnp.float32),
            pltpu.VMEM((C, N), jnp.float32),
            pltpu.VMEM((H, D), jnp.float32),
            pltpu.VMEM((1, H), jnp.float32),
            pltpu.VMEM((C, H), jnp.float32),
            pltpu.VMEM((1, C), jnp.float32),
            pltpu.SemaphoreType.DMA((nc,)),
            pltpu.SemaphoreType.DMA((1,)),
            pltpu.SemaphoreType.DMA((4,)),
        ],
    )(x, W1, b1.reshape(1, H), W2, b2.reshape(1, C))
    return out_t.T


# speedup vs baseline: 1.2176x; 1.2176x over previous
import functools

import jax
import jax.numpy as jnp
from jax.experimental import pallas as pl
from jax.experimental.pallas import tpu as pltpu


def _chunk_compute(xx, w1b, b1, w2b, b2, ones):
    h = jax.lax.dot_general(
        xx.astype(jnp.bfloat16), w1b, (((1,), (1,)), ((), ())),
        preferred_element_type=jnp.float32,
    )
    h = (h + b1).astype(jnp.bfloat16)
    h = jnp.where(h > 0, h, jnp.exp(h) - 1.0)
    h = jax.lax.dot_general(
        h, w2b, (((1,), (1,)), ((), ())),
        preferred_element_type=jnp.float32,
    ) + b2
    h = jnp.where(h > 0, h, jnp.exp(h) - 1.0)
    m = jnp.max(h, axis=1, keepdims=True)
    s = h - m
    e = jnp.exp(s).astype(jnp.bfloat16)
    sums = jax.lax.dot_general(
        e, ones, (((1,), (0,)), ((), ())),
        preferred_element_type=jnp.float32,
    )
    lse = jnp.log(sums[:, :1])
    return s - lse


def _mlp_kernel(
    x_h, w1_h, b1_h, w2_h, b2_h, o_h,
    xbuf, obuf, w1_v, b1_v, w2_v, b2_v, sx, so, sw,
    *, chunks,
):
    wc = [
        pltpu.make_async_copy(w1_h, w1_v, sw.at[0]),
        pltpu.make_async_copy(b1_h, b1_v, sw.at[1]),
        pltpu.make_async_copy(w2_h, w2_v, sw.at[2]),
        pltpu.make_async_copy(b2_h, b2_v, sw.at[3]),
    ]
    for c in wc:
        c.start()
    xc = [
        pltpu.make_async_copy(
            x_h.at[pl.ds(off, sz), :], xbuf.at[i, pl.ds(0, sz)], sx.at[i]
        )
        for i, (off, sz) in enumerate(chunks)
    ]
    for c in xc:
        c.start()
    for c in wc:
        c.wait()
    w1b = w1_v[...].astype(jnp.bfloat16)
    w2b = w2_v[...].astype(jnp.bfloat16)
    b1 = b1_v[...]
    b2 = b2_v[...]
    ones = jnp.ones((64, 128), dtype=jnp.bfloat16)
    for c in xc:
        c.wait()
    for i, (off, sz) in enumerate(chunks):
        r = _chunk_compute(xbuf[i, :sz], w1b, b1, w2b, b2, ones)
        obuf[:, pl.ds(off, sz)] = r.T
    oc = pltpu.make_async_copy(obuf, o_h, so.at[0])
    oc.start()
    oc.wait()


def kernel(x, edge_index, W1, b1, W2, b2):
    N, D = x.shape
    H = W1.shape[0]
    C = W2.shape[0]
    CH = 1024
    chunks = []
    off = 0
    while off < N:
        sz = min(CH, N - off)
        chunks.append((off, sz))
        off += sz
    nc = len(chunks)
    hbm = pl.BlockSpec(memory_space=pltpu.MemorySpace.HBM)
    out_t = pl.pallas_call(
        functools.partial(_mlp_kernel, chunks=tuple(chunks)),
        in_specs=[hbm] * 5,
        out_specs=hbm,
        out_shape=jax.ShapeDtypeStruct((C, N), jnp.float32),
        scratch_shapes=[
            pltpu.VMEM((nc, CH, D), jnp.float32),
            pltpu.VMEM((C, N), jnp.float32),
            pltpu.VMEM((H, D), jnp.float32),
            pltpu.VMEM((1, H), jnp.float32),
            pltpu.VMEM((C, H), jnp.float32),
            pltpu.VMEM((1, C), jnp.float32),
            pltpu.SemaphoreType.DMA((nc,)),
            pltpu.SemaphoreType.DMA((1,)),
            pltpu.SemaphoreType.DMA((4,)),
        ],
    )(x, W1, b1.reshape(1, H), W2, b2.reshape(1, C))
    return out_t.T
